# baseline (speedup 1.0000x reference)
import functools

import jax
import jax.numpy as jnp
from jax import lax
from jax.experimental import pallas as pl
from jax.experimental.pallas import tpu as pltpu
from jax.experimental.pallas import tpu_sc as plsc

TOTAL = 32768
NSEG = 16
NC = 2
NS = 16
NW = NC * NS
ROWS = 8
COLS = 128
PER_TILE = ROWS * COLS
VOCAB = 100000
TCHUNK0 = 6248
TCHUNK = 6280


@functools.partial(
    pl.kernel,
    out_type=(
        jax.ShapeDtypeStruct((NW, NSEG), jnp.float32),
        jax.ShapeDtypeStruct((NW, NSEG), jnp.float32),
    ),
    mesh=plsc.VectorSubcoreMesh(
        core_axis_name="c", subcore_axis_name="s", num_cores=NC, num_subcores=NS
    ),
    compiler_params=pltpu.CompilerParams(needs_layout_passes=False),
    scratch_types=(
        pltpu.VMEM((2 * ROWS, COLS), jnp.int32),
        pltpu.VMEM((PER_TILE,), jnp.int32),
        pltpu.VMEM((ROWS, COLS), jnp.float32),
        pltpu.VMEM((ROWS, COLS), jnp.float32),
        pltpu.VMEM((NSEG,), jnp.float32),
        pltpu.VMEM((NSEG,), jnp.float32),
        pltpu.VMEM_SHARED((VOCAB,), jnp.float32),
        pltpu.VMEM((TCHUNK,), jnp.float32),
        pltpu.SemaphoreType.DMA((ROWS,)),
        pltpu.SemaphoreType.DMA((ROWS,)),
        pltpu.SemaphoreType.DMA,
        pltpu.SemaphoreType.DMA,
    ),
)
def _sc_pool(vec_hbm, seg_hbm, ftab_hbm, stab_hbm,
             num_hbm, den_hbm,
             vidx_v, seg_v, fval_v, sval_v,
             accn_v, accd_v, ftab_s, tmp0_v, fsems, ssems, sem_t0, sem_t1):
    wid = lax.axis_index("s") * NC + lax.axis_index("c")
    sid = lax.axis_index("s")

    tsl = pl.ds(TCHUNK0 * sid, TCHUNK)
    tcp0 = pltpu.async_copy(ftab_hbm.at[tsl], tmp0_v, sem_t0)

    pltpu.sync_copy(vec_hbm.at[pl.ds(2 * ROWS * wid, 2 * ROWS)], vidx_v)
    pltpu.sync_copy(seg_hbm.at[pl.ds(PER_TILE * wid, PER_TILE)], seg_v)

    lane = lax.iota(jnp.int32, 16)

    site_copies = [
        pltpu.async_copy(stab_hbm.at[vidx_v.at[2 * j]], sval_v.at[j], ssems.at[j])
        for j in range(ROWS)
    ]

    tcp0.wait()
    bounce = pltpu.async_copy(tmp0_v, ftab_s.at[tsl], sem_t1)
    bounce.wait()
    plsc.subcore_barrier()

    copies = []
    for j in range(ROWS):
        copies.append((
            pltpu.async_copy(ftab_s.at[vidx_v.at[2 * j + 1]], fval_v.at[j], fsems.at[j]),
            site_copies[j],
        ))

    accn_v[...] = jnp.zeros((NSEG,), jnp.float32)
    accd_v[...] = jnp.zeros((NSEG,), jnp.float32)

    lane_lt15 = lane < 15
    lane_eq15 = lane == 15

    for j in range(ROWS):
        cpf, cps = copies[j]
        cpf.wait()
        cps.wait()

        def body(k, _, j=j):
            pos = j * COLS + k * 16
            pi = lane + pos
            sl = pl.ds(k * 16, 16)
            f = fval_v[j, sl]
            s = sval_v[j, sl]
            g = seg_v[pl.ds(pos, 16)]
            gn = plsc.load_gather(seg_v, [jnp.minimum(pi + 1, PER_TILE - 1)])
            a = jnp.exp(f)
            w = a * s
            ca = plsc.cumsum(a)
            cw = plsc.cumsum(w)
            m = g != gn
            mf = m | lane_eq15
            mm = m & lane_lt15
            plsc.addupdate_scatter(accd_v, [g], ca, mask=mf)
            plsc.addupdate_scatter(accn_v, [g], cw, mask=mf)
            plsc.addupdate_scatter(accd_v, [gn], -ca, mask=mm)
            plsc.addupdate_scatter(accn_v, [gn], -cw, mask=mm)
            return 0

        lax.fori_loop(0, COLS // 16, body, 0, unroll=2)

    pltpu.sync_copy(accn_v, num_hbm.at[wid])
    pltpu.sync_copy(accd_v, den_hbm.at[wid])


def _combine_body(num_ref, den_ref, bias_ref, out_ref):
    num = jnp.sum(num_ref[...], axis=0)
    den = jnp.sum(den_ref[...], axis=0) + 0.001
    out_ref[...] = num / den + bias_ref[0]


_combine = pl.pallas_call(
    _combine_body,
    out_shape=jax.ShapeDtypeStruct((NSEG,), jnp.float32),
)


def kernel(vectors, segment_ids, frag_table, site_table, bias):
    vec = vectors.reshape(TOTAL // COLS, COLS, 2).transpose(0, 2, 1)
    vec = vec.reshape(2 * TOTAL // COLS, COLS)
    num_parts, den_parts = _sc_pool(
        vec, segment_ids, frag_table.reshape(-1), site_table.reshape(-1)
    )
    return _combine(num_parts, den_parts, bias)


# speedup vs baseline: 1.0065x; 1.0065x over previous
import functools

import jax
import jax.numpy as jnp
from jax import lax
from jax.experimental import pallas as pl
from jax.experimental.pallas import tpu as pltpu
from jax.experimental.pallas import tpu_sc as plsc

TOTAL = 32768
NSEG = 16
NC = 2
NS = 16
NW = NC * NS
ROWS = 8
COLS = 128
PER_TILE = ROWS * COLS
VOCAB = 100000
TCHUNK0 = 6248
TCHUNK = 6280


@functools.partial(
    pl.kernel,
    out_type=(
        jax.ShapeDtypeStruct((NW, NSEG), jnp.float32),
        jax.ShapeDtypeStruct((NW, NSEG), jnp.float32),
    ),
    mesh=plsc.VectorSubcoreMesh(
        core_axis_name="c", subcore_axis_name="s", num_cores=NC, num_subcores=NS
    ),
    compiler_params=pltpu.CompilerParams(needs_layout_passes=False),
    scratch_types=(
        pltpu.VMEM((2 * ROWS, COLS), jnp.int32),
        pltpu.VMEM((PER_TILE,), jnp.int32),
        pltpu.VMEM((ROWS, COLS), jnp.float32),
        pltpu.VMEM((ROWS, COLS), jnp.float32),
        pltpu.VMEM((NSEG,), jnp.float32),
        pltpu.VMEM((NSEG,), jnp.float32),
        pltpu.VMEM_SHARED((VOCAB,), jnp.float32),
        pltpu.VMEM_SHARED((VOCAB,), jnp.float32),
        pltpu.VMEM((TCHUNK,), jnp.float32),
        pltpu.VMEM((TCHUNK,), jnp.float32),
        pltpu.SemaphoreType.DMA((ROWS,)),
        pltpu.SemaphoreType.DMA((ROWS,)),
        pltpu.SemaphoreType.DMA,
        pltpu.SemaphoreType.DMA,
    ),
)
def _sc_pool(vec_hbm, seg_hbm, ftab_hbm, stab_hbm,
             num_hbm, den_hbm,
             vidx_v, seg_v, fval_v, sval_v,
             accn_v, accd_v, ftab_s, stab_s, tmp0_v, tmp1_v,
             fsems, ssems, sem_t0, sem_t1):
    wid = lax.axis_index("s") * NC + lax.axis_index("c")
    sid = lax.axis_index("s")

    tsl = pl.ds(TCHUNK0 * sid, TCHUNK)
    tcp0 = pltpu.async_copy(ftab_hbm.at[tsl], tmp0_v, sem_t0)
    tcp1 = pltpu.async_copy(stab_hbm.at[tsl], tmp1_v, sem_t1)

    pltpu.sync_copy(vec_hbm.at[pl.ds(2 * ROWS * wid, 2 * ROWS)], vidx_v)

    lane = lax.iota(jnp.int32, 16)

    tcp0.wait()
    bounce0 = pltpu.async_copy(tmp0_v, ftab_s.at[tsl], sem_t0)
    tcp1.wait()
    bounce1 = pltpu.async_copy(tmp1_v, stab_s.at[tsl], sem_t1)
    bounce0.wait()
    plsc.subcore_barrier()

    fcopies = [
        pltpu.async_copy(ftab_s.at[vidx_v.at[2 * j + 1]], fval_v.at[j], fsems.at[j])
        for j in range(ROWS)
    ]
    bounce1.wait()
    plsc.subcore_barrier()
    scopies = [
        pltpu.async_copy(stab_s.at[vidx_v.at[2 * j]], sval_v.at[j], ssems.at[j])
        for j in range(ROWS)
    ]
    copies = list(zip(fcopies, scopies))

    pltpu.sync_copy(seg_hbm.at[pl.ds(PER_TILE * wid, PER_TILE)], seg_v)

    accn_v[...] = jnp.zeros((NSEG,), jnp.float32)
    accd_v[...] = jnp.zeros((NSEG,), jnp.float32)

    lane_lt15 = lane < 15
    lane_eq15 = lane == 15

    for j in range(ROWS):
        cpf, cps = copies[j]
        cpf.wait()
        cps.wait()

        def body(k, _, j=j):
            pos = j * COLS + k * 16
            pi = lane + pos
            sl = pl.ds(k * 16, 16)
            f = fval_v[j, sl]
            s = sval_v[j, sl]
            g = seg_v[pl.ds(pos, 16)]
            gn = plsc.load_gather(seg_v, [jnp.minimum(pi + 1, PER_TILE - 1)])
            a = jnp.exp(f)
            w = a * s
            ca = plsc.cumsum(a)
            cw = plsc.cumsum(w)
            m = g != gn
            mf = m | lane_eq15
            mm = m & lane_lt15
            plsc.addupdate_scatter(accd_v, [g], ca, mask=mf)
            plsc.addupdate_scatter(accn_v, [g], cw, mask=mf)
            plsc.addupdate_scatter(accd_v, [gn], -ca, mask=mm)
            plsc.addupdate_scatter(accn_v, [gn], -cw, mask=mm)
            return 0

        lax.fori_loop(0, COLS // 16, body, 0, unroll=2)

    pltpu.sync_copy(accn_v, num_hbm.at[wid])
    pltpu.sync_copy(accd_v, den_hbm.at[wid])


def _combine_body(num_ref, den_ref, bias_ref, out_ref):
    num = jnp.sum(num_ref[...], axis=0)
    den = jnp.sum(den_ref[...], axis=0) + 0.001
    out_ref[...] = num / den + bias_ref[0]


_combine = pl.pallas_call(
    _combine_body,
    out_shape=jax.ShapeDtypeStruct((NSEG,), jnp.float32),
)


def kernel(vectors, segment_ids, frag_table, site_table, bias):
    vec = vectors.reshape(TOTAL // COLS, COLS, 2).transpose(0, 2, 1)
    vec = vec.reshape(2 * TOTAL // COLS, COLS)
    num_parts, den_parts = _sc_pool(
        vec, segment_ids, frag_table.reshape(-1), site_table.reshape(-1)
    )
    return _combine(num_parts, den_parts, bias)


# speedup vs baseline: 1.0086x; 1.0020x over previous
import functools

import jax
import jax.numpy as jnp
from jax import lax
from jax.experimental import pallas as pl
from jax.experimental.pallas import tpu as pltpu
from jax.experimental.pallas import tpu_sc as plsc

TOTAL = 32768
NSEG = 16
NC = 2
NS = 16
NW = NC * NS
ROWS = 8
COLS = 128
PER_TILE = ROWS * COLS
VOCAB = 100000
TCHUNK0 = 6248
TCHUNK = 6280


@functools.partial(
    pl.kernel,
    out_type=(
        jax.ShapeDtypeStruct((NW, NSEG), jnp.float32),
        jax.ShapeDtypeStruct((NW, NSEG), jnp.float32),
    ),
    mesh=plsc.VectorSubcoreMesh(
        core_axis_name="c", subcore_axis_name="s", num_cores=NC, num_subcores=NS
    ),
    compiler_params=pltpu.CompilerParams(
        needs_layout_passes=False, skip_device_barrier=True
    ),
    scratch_types=(
        pltpu.VMEM((2 * ROWS, COLS), jnp.int32),
        pltpu.VMEM((PER_TILE,), jnp.int32),
        pltpu.VMEM((ROWS, COLS), jnp.float32),
        pltpu.VMEM((ROWS, COLS), jnp.float32),
        pltpu.VMEM((NSEG,), jnp.float32),
        pltpu.VMEM((NSEG,), jnp.float32),
        pltpu.VMEM_SHARED((VOCAB,), jnp.float32),
        pltpu.VMEM_SHARED((VOCAB,), jnp.float32),
        pltpu.VMEM((TCHUNK,), jnp.float32),
        pltpu.VMEM((TCHUNK,), jnp.float32),
        pltpu.SemaphoreType.DMA((ROWS,)),
        pltpu.SemaphoreType.DMA,
        pltpu.SemaphoreType.DMA,
    ),
)
def _sc_pool(vec_hbm, seg_hbm, ftab_hbm, stab_hbm,
             num_hbm, den_hbm,
             vidx_v, seg_v, fval_v, sval_v,
             accn_v, accd_v, ftab_s, stab_s, tmp0_v, tmp1_v, sems, sem_t0, sem_t1):
    wid = lax.axis_index("s") * NC + lax.axis_index("c")
    sid = lax.axis_index("s")

    tsl = pl.ds(TCHUNK0 * sid, TCHUNK)
    tcp0 = pltpu.async_copy(ftab_hbm.at[tsl], tmp0_v, sem_t0)
    tcp1 = pltpu.async_copy(stab_hbm.at[tsl], tmp1_v, sem_t1)

    pltpu.sync_copy(vec_hbm.at[pl.ds(2 * ROWS * wid, 2 * ROWS)], vidx_v)
    pltpu.sync_copy(seg_hbm.at[pl.ds(PER_TILE * wid, PER_TILE)], seg_v)

    lane = lax.iota(jnp.int32, 16)

    tcp0.wait()
    pltpu.sync_copy(tmp0_v, ftab_s.at[tsl])
    tcp1.wait()
    pltpu.sync_copy(tmp1_v, stab_s.at[tsl])
    plsc.subcore_barrier()

    copies = []
    for j in range(ROWS):
        copies.append((
            pltpu.async_copy(ftab_s.at[vidx_v.at[2 * j + 1]], fval_v.at[j], sems.at[j]),
            pltpu.async_copy(stab_s.at[vidx_v.at[2 * j]], sval_v.at[j], sems.at[j]),
        ))

    accn_v[...] = jnp.zeros((NSEG,), jnp.float32)
    accd_v[...] = jnp.zeros((NSEG,), jnp.float32)

    lane_lt15 = lane < 15
    lane_eq15 = lane == 15

    for j in range(ROWS):
        cpf, cps = copies[j]
        cpf.wait()
        cps.wait()

        def body(k, _, j=j):
            pos = j * COLS + k * 16
            pi = lane + pos
            sl = pl.ds(k * 16, 16)
            f = fval_v[j, sl]
            s = sval_v[j, sl]
            g = seg_v[pl.ds(pos, 16)]
            gn = plsc.load_gather(seg_v, [jnp.minimum(pi + 1, PER_TILE - 1)])
            a = jnp.exp(f)
            w = a * s
            ca = plsc.cumsum(a)
            cw = plsc.cumsum(w)
            m = g != gn
            mf = m | lane_eq15
            mm = m & lane_lt15
            plsc.addupdate_scatter(accd_v, [g], ca, mask=mf)
            plsc.addupdate_scatter(accn_v, [g], cw, mask=mf)
            plsc.addupdate_scatter(accd_v, [gn], -ca, mask=mm)
            plsc.addupdate_scatter(accn_v, [gn], -cw, mask=mm)
            return 0

        lax.fori_loop(0, COLS // 16, body, 0, unroll=2)

    pltpu.sync_copy(accn_v, num_hbm.at[wid])
    pltpu.sync_copy(accd_v, den_hbm.at[wid])


def _combine_body(num_ref, den_ref, bias_ref, out_ref):
    num = jnp.sum(num_ref[...], axis=0)
    den = jnp.sum(den_ref[...], axis=0) + 0.001
    out_ref[...] = num / den + bias_ref[0]


_combine = pl.pallas_call(
    _combine_body,
    out_shape=jax.ShapeDtypeStruct((NSEG,), jnp.float32),
)


def kernel(vectors, segment_ids, frag_table, site_table, bias):
    vec = vectors.reshape(TOTAL // COLS, COLS, 2).transpose(0, 2, 1)
    vec = vec.reshape(2 * TOTAL // COLS, COLS)
    num_parts, den_parts = _sc_pool(
        vec, segment_ids, frag_table.reshape(-1), site_table.reshape(-1)
    )
    return _combine(num_parts, den_parts, bias)


# speedup vs baseline: 1.0091x; 1.0006x over previous
import functools

import jax
import jax.numpy as jnp
from jax import lax
from jax.experimental import pallas as pl
from jax.experimental.pallas import tpu as pltpu
from jax.experimental.pallas import tpu_sc as plsc

TOTAL = 32768
NSEG = 16
NC = 2
NS = 16
NW = NC * NS
ROWS = 8
COLS = 128
PER_TILE = ROWS * COLS
VOCAB = 100000
TCHUNK0 = 6248
TCHUNK = 6280


@functools.partial(
    pl.kernel,
    out_type=(
        jax.ShapeDtypeStruct((NW, NSEG), jnp.float32),
        jax.ShapeDtypeStruct((NW, NSEG), jnp.float32),
    ),
    mesh=plsc.VectorSubcoreMesh(
        core_axis_name="c", subcore_axis_name="s", num_cores=NC, num_subcores=NS
    ),
    compiler_params=pltpu.CompilerParams(needs_layout_passes=False),
    scratch_types=(
        pltpu.VMEM((2 * ROWS, COLS), jnp.int32),
        pltpu.VMEM((PER_TILE,), jnp.int32),
        pltpu.VMEM((ROWS, COLS), jnp.float32),
        pltpu.VMEM((ROWS, COLS), jnp.float32),
        pltpu.VMEM((NSEG,), jnp.float32),
        pltpu.VMEM((NSEG,), jnp.float32),
        pltpu.VMEM_SHARED((VOCAB,), jnp.float32),
        pltpu.VMEM_SHARED((VOCAB,), jnp.float32),
        pltpu.VMEM((TCHUNK,), jnp.float32),
        pltpu.VMEM((TCHUNK,), jnp.float32),
        pltpu.SemaphoreType.DMA((ROWS,)),
        pltpu.SemaphoreType.DMA,
        pltpu.SemaphoreType.DMA,
    ),
)
def _sc_pool(vec_hbm, seg_hbm, ftab_hbm, stab_hbm,
             num_hbm, den_hbm,
             vidx_v, seg_v, fval_v, sval_v,
             accn_v, accd_v, ftab_s, stab_s, tmp0_v, tmp1_v, sems, sem_t0, sem_t1):
    wid = lax.axis_index("s") * NC + lax.axis_index("c")
    sid = lax.axis_index("s")

    tsl = pl.ds(TCHUNK0 * sid, TCHUNK)
    tcp0 = pltpu.async_copy(ftab_hbm.at[tsl], tmp0_v, sem_t0)
    tcp1 = pltpu.async_copy(stab_hbm.at[tsl], tmp1_v, sem_t1)

    pltpu.sync_copy(vec_hbm.at[pl.ds(2 * ROWS * wid, 2 * ROWS)], vidx_v)
    pltpu.sync_copy(seg_hbm.at[pl.ds(PER_TILE * wid, PER_TILE)], seg_v)

    lane = lax.iota(jnp.int32, 16)

    tcp0.wait()
    pltpu.sync_copy(tmp0_v, ftab_s.at[tsl])
    tcp1.wait()
    pltpu.sync_copy(tmp1_v, stab_s.at[tsl])
    plsc.subcore_barrier()

    copies = []
    for j in range(ROWS):
        copies.append((
            pltpu.async_copy(ftab_s.at[vidx_v.at[2 * j + 1]], fval_v.at[j], sems.at[j]),
            pltpu.async_copy(stab_s.at[vidx_v.at[2 * j]], sval_v.at[j], sems.at[j]),
        ))

    accn_v[...] = jnp.zeros((NSEG,), jnp.float32)
    accd_v[...] = jnp.zeros((NSEG,), jnp.float32)

    lane_lt15 = lane < 15
    lane_eq15 = lane == 15

    for j in range(ROWS):
        cpf, cps = copies[j]
        cpf.wait()
        cps.wait()

        def body(k, _, j=j):
            pos = j * COLS + k * 16
            pi = lane + pos
            sl = pl.ds(k * 16, 16)
            f = fval_v[j, sl]
            s = sval_v[j, sl]
            g = seg_v[pl.ds(pos, 16)]
            gn = plsc.load_gather(seg_v, [jnp.minimum(pi + 1, PER_TILE - 1)])
            a = jnp.exp(f)
            w = a * s
            ca = plsc.cumsum(a)
            cw = plsc.cumsum(w)
            m = g != gn
            mf = m | lane_eq15
            mm = m & lane_lt15
            plsc.addupdate_scatter(accd_v, [g], ca, mask=mf)
            plsc.addupdate_scatter(accn_v, [g], cw, mask=mf)
            plsc.addupdate_scatter(accd_v, [gn], -ca, mask=mm)
            plsc.addupdate_scatter(accn_v, [gn], -cw, mask=mm)
            return 0

        lax.fori_loop(0, COLS // 16, body, 0, unroll=2)

    pltpu.sync_copy(accn_v, num_hbm.at[wid])
    pltpu.sync_copy(accd_v, den_hbm.at[wid])


def _combine_body(num_ref, den_ref, bias_ref, out_ref):
    num = jnp.sum(num_ref[...], axis=0)
    den = jnp.sum(den_ref[...], axis=0) + 0.001
    out_ref[...] = num / den + bias_ref[0]


_combine = pl.pallas_call(
    _combine_body,
    out_shape=jax.ShapeDtypeStruct((NSEG,), jnp.float32),
)


def kernel(vectors, segment_ids, frag_table, site_table, bias):
    vec = vectors.reshape(TOTAL // COLS, COLS, 2).transpose(0, 2, 1)
    vec = vec.reshape(2 * TOTAL // COLS, COLS)
    num_parts, den_parts = _sc_pool(
        vec, segment_ids, frag_table.reshape(-1), site_table.reshape(-1)
    )
    return _combine(num_parts, den_parts, bias)


# speedup vs baseline: 1.0122x; 1.0031x over previous
import functools

import jax
import jax.numpy as jnp
from jax import lax
from jax.experimental import pallas as pl
from jax.experimental.pallas import tpu as pltpu
from jax.experimental.pallas import tpu_sc as plsc

TOTAL = 32768
NSEG = 16
NC = 2
NS = 16
NW = NC * NS
ROWS = 8
COLS = 128
PER_TILE = ROWS * COLS
VOCAB = 100000
TCHUNK0 = 6248
TCHUNK = 6280


@functools.partial(
    pl.kernel,
    out_type=(
        jax.ShapeDtypeStruct((NW, NSEG), jnp.float32),
        jax.ShapeDtypeStruct((NW, NSEG), jnp.float32),
    ),
    mesh=plsc.VectorSubcoreMesh(
        core_axis_name="c", subcore_axis_name="s", num_cores=NC, num_subcores=NS
    ),
    compiler_params=pltpu.CompilerParams(needs_layout_passes=False),
    scratch_types=(
        pltpu.VMEM((2 * ROWS, COLS), jnp.int32),
        pltpu.VMEM((PER_TILE,), jnp.int32),
        pltpu.VMEM((ROWS, COLS), jnp.float32),
        pltpu.VMEM((ROWS, COLS), jnp.float32),
        pltpu.VMEM((NSEG,), jnp.float32),
        pltpu.VMEM((NSEG,), jnp.float32),
        pltpu.VMEM_SHARED((VOCAB,), jnp.float32),
        pltpu.VMEM_SHARED((VOCAB,), jnp.float32),
        pltpu.VMEM((TCHUNK,), jnp.float32),
        pltpu.VMEM((TCHUNK,), jnp.float32),
        pltpu.SemaphoreType.DMA((ROWS,)),
        pltpu.SemaphoreType.DMA,
        pltpu.SemaphoreType.DMA,
    ),
)
def _sc_pool(vec_hbm, seg_hbm, ftab_hbm, stab_hbm,
             num_hbm, den_hbm,
             vidx_v, seg_v, fval_v, sval_v,
             accn_v, accd_v, ftab_s, stab_s, tmp0_v, tmp1_v, sems, sem_t0, sem_t1):
    wid = lax.axis_index("s") * NC + lax.axis_index("c")
    sid = lax.axis_index("s")

    tsl = pl.ds(TCHUNK0 * sid, TCHUNK)
    tcp0 = pltpu.async_copy(ftab_hbm.at[tsl], tmp0_v, sem_t0)
    tcp1 = pltpu.async_copy(stab_hbm.at[tsl], tmp1_v, sem_t1)

    pltpu.sync_copy(vec_hbm.at[pl.ds(2 * ROWS * wid, 2 * ROWS)], vidx_v)
    pltpu.sync_copy(seg_hbm.at[pl.ds(PER_TILE * wid, PER_TILE)], seg_v)

    lane = lax.iota(jnp.int32, 16)

    tcp0.wait()
    pltpu.sync_copy(tmp0_v, ftab_s.at[tsl])
    tcp1.wait()
    pltpu.sync_copy(tmp1_v, stab_s.at[tsl])
    plsc.subcore_barrier()

    copies = []
    for j in range(ROWS):
        copies.append((
            pltpu.async_copy(ftab_s.at[vidx_v.at[2 * j + 1]], fval_v.at[j], sems.at[j]),
            pltpu.async_copy(stab_s.at[vidx_v.at[2 * j]], sval_v.at[j], sems.at[j]),
        ))

    accn_v[...] = jnp.zeros((NSEG,), jnp.float32)
    accd_v[...] = jnp.zeros((NSEG,), jnp.float32)

    lane_lt15 = lane < 15
    lane_eq15 = lane == 15

    for j in range(ROWS):
        cpf, cps = copies[j]
        cpf.wait()
        cps.wait()

        def body(k, _, j=j):
            pos = j * COLS + k * 16
            pi = lane + pos
            sl = pl.ds(k * 16, 16)
            f = fval_v[j, sl]
            s = sval_v[j, sl]
            g = seg_v[pl.ds(pos, 16)]
            gn = plsc.load_gather(seg_v, [jnp.minimum(pi + 1, PER_TILE - 1)])
            a = jnp.exp(f)
            w = a * s
            ca = plsc.cumsum(a)
            cw = plsc.cumsum(w)
            m = g != gn
            mf = m | lane_eq15
            mm = m & lane_lt15
            plsc.addupdate_scatter(accd_v, [g], ca, mask=mf)
            plsc.addupdate_scatter(accn_v, [g], cw, mask=mf)
            plsc.addupdate_scatter(accd_v, [gn], -ca, mask=mm)
            plsc.addupdate_scatter(accn_v, [gn], -cw, mask=mm)
            return 0

        lax.fori_loop(0, COLS // 16, body, 0, unroll=4)

    pltpu.sync_copy(accn_v, num_hbm.at[wid])
    pltpu.sync_copy(accd_v, den_hbm.at[wid])


def _combine_body(num_ref, den_ref, bias_ref, out_ref):
    num = jnp.sum(num_ref[...], axis=0)
    den = jnp.sum(den_ref[...], axis=0) + 0.001
    out_ref[...] = num / den + bias_ref[0]


_combine = pl.pallas_call(
    _combine_body,
    out_shape=jax.ShapeDtypeStruct((NSEG,), jnp.float32),
)


def kernel(vectors, segment_ids, frag_table, site_table, bias):
    vec = vectors.reshape(TOTAL // COLS, COLS, 2).transpose(0, 2, 1)
    vec = vec.reshape(2 * TOTAL // COLS, COLS)
    num_parts, den_parts = _sc_pool(
        vec, segment_ids, frag_table.reshape(-1), site_table.reshape(-1)
    )
    return _combine(num_parts, den_parts, bias)
